# baseline (speedup 1.0000x reference)
import jax
import jax.numpy as jnp
from jax.experimental import pallas as pl
from jax.experimental.pallas import tpu as pltpu

_ROWS = 128
_COLS = 100000
_SUP_STRIDE = 200
_SUP_LIMIT = 6400
_N_CHUNKS = 32
_CHUNK_ROWS = _ROWS // _N_CHUNKS


def _chunk_slice(x_hbm, i):
    return x_hbm.at[pl.ds(i * _CHUNK_ROWS, _CHUNK_ROWS), :]


def _body(x_hbm, o_hbm, bufs, sem_in, sem_out):
    for i in range(_N_CHUNKS):
        pltpu.make_async_copy(
            _chunk_slice(x_hbm, i), bufs.at[i], sem_in.at[i]
        ).start()
    neg = jnp.full((_CHUNK_ROWS, 1), -jnp.inf, jnp.float32)
    for i in range(_N_CHUNKS):
        pltpu.make_async_copy(
            _chunk_slice(x_hbm, i), bufs.at[i], sem_in.at[i]
        ).wait()
        for c in range(0, _SUP_LIMIT, _SUP_STRIDE):
            bufs[i, :, c : c + 1] = neg
        pltpu.make_async_copy(
            bufs.at[i], _chunk_slice(o_hbm, i), sem_out.at[i]
        ).start()
    for i in range(_N_CHUNKS):
        pltpu.make_async_copy(
            bufs.at[i], _chunk_slice(o_hbm, i), sem_out.at[i]
        ).wait()


def kernel(scores):
    return pl.pallas_call(
        _body,
        in_specs=[pl.BlockSpec(memory_space=pl.MemorySpace.ANY)],
        out_specs=pl.BlockSpec(memory_space=pl.MemorySpace.ANY),
        out_shape=jax.ShapeDtypeStruct((_ROWS, _COLS), scores.dtype),
        scratch_shapes=[
            pltpu.MemorySpace.VMEM((_N_CHUNKS, _CHUNK_ROWS, _COLS), jnp.float32),
            pltpu.SemaphoreType.DMA((_N_CHUNKS,)),
            pltpu.SemaphoreType.DMA((_N_CHUNKS,)),
        ],
    )(scores)


# speedup vs baseline: 1.3074x; 1.3074x over previous
import jax
import jax.numpy as jnp
from jax.experimental import pallas as pl

_ROWS = 128
_COLS = 100000
_SUP_STRIDE = 200
_SUP_LIMIT = 6400
_BW = 3200


def _scatter_body(x_ref, o_ref):
    o_ref[...] = x_ref[...]
    neg = jnp.full((_ROWS, 1), -jnp.inf, jnp.float32)
    for c in range(0, _BW, _SUP_STRIDE):
        o_ref[:, c : c + 1] = neg


def kernel(scores):
    return pl.pallas_call(
        _scatter_body,
        grid=(_SUP_LIMIT // _BW,),
        in_specs=[pl.BlockSpec((_ROWS, _BW), lambda i: (0, i))],
        out_specs=pl.BlockSpec((_ROWS, _BW), lambda i: (0, i)),
        out_shape=jax.ShapeDtypeStruct((_ROWS, _COLS), scores.dtype),
        input_output_aliases={0: 0},
    )(scores)
